# baseline (device time: 17966 ns/iter reference)
import jax
import jax.numpy as jnp
from jax import lax
from jax.experimental import pallas as pl
from jax.experimental.pallas import tpu as pltpu

N_DEV = 4
EPS = 1e-5
C = 8
D = 4


def kernel(x, gamma):
    m, n_per = x.shape
    n_global = n_per * N_DEV
    r = m // 128
    rows_c = m // C
    rp = r // C
    gamma2d = gamma.reshape(1, n_per)

    def body(x_hbm, g_ref, out_hbm, x_vmem, out_vmem, src_ref, comm_ref,
             in_sems, out_sems, send_sems, recv_sems):
        my = lax.axis_index("i")

        in_copies = []
        for c in range(C):
            cp = pltpu.make_async_copy(
                x_hbm.at[pl.ds(c * rows_c, rows_c), :],
                x_vmem.at[pl.ds(c * rows_c, rows_c), :],
                in_sems.at[c],
            )
            cp.start()
            in_copies.append(cp)

        barrier_sem = pltpu.get_barrier_semaphore()
        for k in range(1, N_DEV):
            pl.semaphore_signal(
                barrier_sem, inc=1,
                device_id=(lax.rem(my + k, N_DEV),),
                device_id_type=pl.DeviceIdType.MESH,
            )
        pl.semaphore_wait(barrier_sem, N_DEV - 1)

        g = g_ref[0, :][None, None, :]
        rdmas = [[None] * (N_DEV - 1) for _ in range(C)]
        out_copies = []

        def compute_and_send(c):
            in_copies[c].wait()
            xc = x_vmem[pl.ds(c * rows_c, rows_c), :].reshape(rp, 128, n_per)
            src_ref[pl.ds(c * rp, rp), :] = jnp.sum(xc * xc, axis=2)
            for j in range(N_DEV - 1):
                rdma = pltpu.make_async_remote_copy(
                    src_ref=src_ref.at[pl.ds(c * rp, rp), :],
                    dst_ref=comm_ref.at[j, pl.ds(c * rp, rp), :],
                    send_sem=send_sems.at[c, j],
                    recv_sem=recv_sems.at[c, j],
                    device_id=(lax.rem(my + j + 1, N_DEV),),
                    device_id_type=pl.DeviceIdType.MESH,
                )
                rdma.start()
                rdmas[c][j] = rdma

        def normalize_and_store(c):
            sl = pl.ds(c * rp, rp)
            acc = src_ref[sl, :]
            for j in range(N_DEV - 1):
                rdmas[c][j].wait_recv()
                acc = acc + comm_ref[j, sl, :]
            inv = lax.rsqrt(acc * (1.0 / n_global) + EPS)
            xc = x_vmem[pl.ds(c * rows_c, rows_c), :].reshape(rp, 128, n_per)
            out_vmem[pl.ds(c * rows_c, rows_c), :] = (
                xc * g * inv[:, :, None]
            ).reshape(rows_c, n_per)
            cp = pltpu.make_async_copy(
                out_vmem.at[pl.ds(c * rows_c, rows_c), :],
                out_hbm.at[pl.ds(c * rows_c, rows_c), :],
                out_sems.at[c],
            )
            cp.start()
            out_copies.append(cp)

        for s in range(C + D):
            if s < C:
                compute_and_send(s)
            if s >= D:
                normalize_and_store(s - D)

        for cp in out_copies:
            cp.wait()
        for c in range(C):
            for j in range(N_DEV - 1):
                rdmas[c][j].wait_send()

    return pl.pallas_call(
        body,
        out_shape=jax.ShapeDtypeStruct((m, n_per), jnp.float32),
        in_specs=[
            pl.BlockSpec(memory_space=pl.ANY),
            pl.BlockSpec(memory_space=pltpu.VMEM),
        ],
        out_specs=pl.BlockSpec(memory_space=pl.ANY),
        scratch_shapes=[
            pltpu.VMEM((m, n_per), jnp.float32),
            pltpu.VMEM((m, n_per), jnp.float32),
            pltpu.VMEM((r, 128), jnp.float32),
            pltpu.VMEM((N_DEV - 1, r, 128), jnp.float32),
            pltpu.SemaphoreType.DMA((C,)),
            pltpu.SemaphoreType.DMA((C,)),
            pltpu.SemaphoreType.DMA((C, N_DEV - 1)),
            pltpu.SemaphoreType.DMA((C, N_DEV - 1)),
        ],
        compiler_params=pltpu.CompilerParams(collective_id=0),
    )(x, gamma2d)


# device time: 16021 ns/iter; 1.1214x vs baseline; 1.1214x over previous
import jax
import jax.numpy as jnp
from jax import lax
from jax.experimental import pallas as pl
from jax.experimental.pallas import tpu as pltpu

N_DEV = 4
EPS = 1e-5


def kernel(x, gamma):
    m, n_per = x.shape
    n_global = n_per * N_DEV
    r = m // 128

    def body(x_hbm, g_ref, out_hbm, x_vmem, out_vmem, src_ref, comm_ref,
             in_sem, out_sem, send_sems, recv_sems):
        my = lax.axis_index("i")

        cp_in = pltpu.make_async_copy(x_hbm, x_vmem, in_sem)
        cp_in.start()

        barrier_sem = pltpu.get_barrier_semaphore()
        for k in range(1, N_DEV):
            pl.semaphore_signal(
                barrier_sem, inc=1,
                device_id=(lax.rem(my + k, N_DEV),),
                device_id_type=pl.DeviceIdType.MESH,
            )
        pl.semaphore_wait(barrier_sem, N_DEV - 1)

        cp_in.wait()
        x3 = x_vmem[...].reshape(r, 128, n_per)
        partial = jnp.sum(x3 * x3, axis=2)
        src_ref[...] = partial

        rdmas = []
        for j in range(N_DEV - 1):
            rdma = pltpu.make_async_remote_copy(
                src_ref=src_ref,
                dst_ref=comm_ref.at[j],
                send_sem=send_sems.at[j],
                recv_sem=recv_sems.at[j],
                device_id=(lax.rem(my + j + 1, N_DEV),),
                device_id_type=pl.DeviceIdType.MESH,
            )
            rdma.start()
            rdmas.append(rdma)

        acc = partial
        for j in range(N_DEV - 1):
            rdmas[j].wait_recv()
            acc = acc + comm_ref[j]

        inv = lax.rsqrt(acc * (1.0 / n_global) + EPS)
        out_vmem[...] = (
            x3 * g_ref[0, :][None, None, :] * inv[:, :, None]
        ).reshape(m, n_per)

        cp_out = pltpu.make_async_copy(out_vmem, out_hbm, out_sem)
        cp_out.start()
        cp_out.wait()
        for j in range(N_DEV - 1):
            rdmas[j].wait_send()

    return pl.pallas_call(
        body,
        out_shape=jax.ShapeDtypeStruct((m, n_per), jnp.float32),
        in_specs=[
            pl.BlockSpec(memory_space=pl.ANY),
            pl.BlockSpec(memory_space=pltpu.VMEM),
        ],
        out_specs=pl.BlockSpec(memory_space=pl.ANY),
        scratch_shapes=[
            pltpu.VMEM((m, n_per), jnp.float32),
            pltpu.VMEM((m, n_per), jnp.float32),
            pltpu.VMEM((r, 128), jnp.float32),
            pltpu.VMEM((N_DEV - 1, r, 128), jnp.float32),
            pltpu.SemaphoreType.DMA,
            pltpu.SemaphoreType.DMA,
            pltpu.SemaphoreType.DMA((N_DEV - 1,)),
            pltpu.SemaphoreType.DMA((N_DEV - 1,)),
        ],
        compiler_params=pltpu.CompilerParams(collective_id=0),
    )(x, gamma.reshape(1, n_per))


# device time: 14275 ns/iter; 1.2586x vs baseline; 1.1223x over previous
import jax
import jax.numpy as jnp
from jax import lax
from jax.experimental import pallas as pl
from jax.experimental.pallas import tpu as pltpu

N_DEV = 4
EPS = 1e-5


def kernel(x, gamma):
    m, n_per = x.shape
    n_global = n_per * N_DEV
    r = m // 128

    def body(x_hbm, g_hbm, out_hbm, x_vmem, out_vmem, g_vmem, src_ref,
             comm_ref, in_sem, g_sem, out_sem, send_sems, recv_sems):
        my = lax.axis_index("i")

        cp_in = pltpu.make_async_copy(x_hbm, x_vmem, in_sem)
        cp_in.start()
        cp_g = pltpu.make_async_copy(g_hbm, g_vmem, g_sem)
        cp_g.start()

        barrier_sem = pltpu.get_barrier_semaphore()
        for k in range(1, N_DEV):
            pl.semaphore_signal(
                barrier_sem, inc=1,
                device_id=(lax.rem(my + k, N_DEV),),
                device_id_type=pl.DeviceIdType.MESH,
            )

        cp_in.wait()
        x3 = x_vmem[...].reshape(r, 128, n_per)
        partial = jnp.sum(x3 * x3, axis=2)
        src_ref[...] = partial

        pl.semaphore_wait(barrier_sem, N_DEV - 1)

        rdmas = []
        for j in range(N_DEV - 1):
            rdma = pltpu.make_async_remote_copy(
                src_ref=src_ref,
                dst_ref=comm_ref.at[j],
                send_sem=send_sems.at[j],
                recv_sem=recv_sems.at[j],
                device_id=(lax.rem(my + j + 1, N_DEV),),
                device_id_type=pl.DeviceIdType.MESH,
            )
            rdma.start()
            rdmas.append(rdma)

        acc = partial
        for j in range(N_DEV - 1):
            rdmas[j].wait_recv()
            acc = acc + comm_ref[j]

        cp_g.wait()
        inv = lax.rsqrt(acc * (1.0 / n_global) + EPS)
        out_vmem[...] = (
            x3 * g_vmem[0, :][None, None, :] * inv[:, :, None]
        ).reshape(m, n_per)

        cp_out = pltpu.make_async_copy(out_vmem, out_hbm, out_sem)
        cp_out.start()
        cp_out.wait()
        for j in range(N_DEV - 1):
            rdmas[j].wait_send()

    call = pl.pallas_call(
        body,
        out_shape=jax.ShapeDtypeStruct((m, n_per), jnp.float32),
        in_specs=[
            pl.BlockSpec(memory_space=pltpu.MemorySpace.HBM),
            pl.BlockSpec(memory_space=pltpu.MemorySpace.HBM),
        ],
        out_specs=pl.BlockSpec(memory_space=pltpu.MemorySpace.HBM),
        scratch_shapes=[
            pltpu.VMEM((m, n_per), jnp.float32),
            pltpu.VMEM((m, n_per), jnp.float32),
            pltpu.VMEM((1, n_per), jnp.float32),
            pltpu.VMEM((r, 128), jnp.float32),
            pltpu.VMEM((N_DEV - 1, r, 128), jnp.float32),
            pltpu.SemaphoreType.DMA,
            pltpu.SemaphoreType.DMA,
            pltpu.SemaphoreType.DMA,
            pltpu.SemaphoreType.DMA((N_DEV - 1,)),
            pltpu.SemaphoreType.DMA((N_DEV - 1,)),
        ],
        compiler_params=pltpu.CompilerParams(collective_id=0),
    )
    xc = pltpu.with_memory_space_constraint(x, pltpu.MemorySpace.HBM)
    gc = pltpu.with_memory_space_constraint(
        gamma.reshape(1, n_per), pltpu.MemorySpace.HBM
    )
    return call(xc, gc)
